# baseline (speedup 1.0000x reference)
import functools

import jax
import jax.numpy as jnp
from jax import lax
from jax.experimental import pallas as pl
from jax.experimental.pallas import tpu as pltpu
from jax.experimental.pallas import tpu_sc as plsc

VOCAB = 1000000
EMBED_DIM = 64
PAD_LEN = 200
BATCH = 4096
CLASS_NUM = 16

_D = EMBED_DIM
_L = PAD_LEN
_NC = 2
_NS = 16
_NW = _NC * _NS
_BW = BATCH // _NW
_NB = 8
_UNROLL = 16


def _fire(table_hbm, tex_v, buf, sem, c):
    pltpu.make_async_copy(table_hbm.at[tex_v.at[c]], buf, sem).start()


def _drain(table_hbm, buf, sem):
    pltpu.make_async_copy(table_hbm.at[pl.ds(0, _BW)], buf, sem).wait()


def _accum(buf, acc_v):
    def body(t, carry):
        for u in range(_UNROLL):
            j = 4 * t + u // 4
            off = 16 * (u % 4)
            plsc.addupdate(acc_v.at[j, pl.ds(off, 16)], buf[j, pl.ds(off, 16)])
        return carry
    lax.fori_loop(0, (_BW * _D) // (16 * _UNROLL), body, 0)


@functools.partial(
    pl.kernel,
    mesh=plsc.VectorSubcoreMesh(core_axis_name="c", subcore_axis_name="s"),
    out_type=jax.ShapeDtypeStruct((BATCH, _D), jnp.float32),
    compiler_params=pltpu.CompilerParams(use_tc_tiling_on_sc=False),
    scratch_types=(
        [pltpu.VMEM((_L, _BW), jnp.int32)]
        + [pltpu.VMEM((_BW, _D), jnp.float32)]
        + [pltpu.VMEM((_BW, _D), jnp.float32)] * _NB
        + [pltpu.SemaphoreType.DMA] * _NB
    ),
)
def _sc_lookup_sum(texts_t_hbm, table_hbm, out_hbm, tex_v, acc_v, *ring):
    bufs, sems = ring[:_NB], ring[_NB:]
    wid = lax.axis_index("s") * _NC + lax.axis_index("c")
    base = wid * _BW
    pltpu.sync_copy(texts_t_hbm.at[wid], tex_v)

    z = jnp.zeros((16,), jnp.float32)

    def zero_body(t, carry):
        for u in range(_UNROLL):
            acc_v[4 * t + u // 4, pl.ds(16 * (u % 4), 16)] = z
        return carry
    lax.fori_loop(0, (_BW * _D) // (16 * _UNROLL), zero_body, 0)

    for b in range(_NB):
        _fire(table_hbm, tex_v, bufs[b], sems[b], b)

    def outer(g, carry):
        for b in range(_NB):
            c = g * _NB + b
            _drain(table_hbm, bufs[b], sems[b])
            _accum(bufs[b], acc_v)

            @pl.when(c + _NB < _L)
            def _():
                _fire(table_hbm, tex_v, bufs[b], sems[b], c + _NB)
        return carry

    lax.fori_loop(0, _L // _NB, outer, 0)
    pltpu.sync_copy(acc_v, out_hbm.at[pl.ds(base, _BW)])


def _fc_body(x_ref, w_ref, b_ref, o_ref):
    o_ref[...] = (
        jnp.dot(x_ref[...], w_ref[...], preferred_element_type=jnp.float32)
        + b_ref[...]
    )


_fc_call = pl.pallas_call(
    _fc_body,
    out_shape=jax.ShapeDtypeStruct((BATCH, 128), jnp.float32),
)


def kernel(texts, table, fc_w, fc_b):
    texts_t = jnp.transpose(
        texts.astype(jnp.int32).reshape(_NW, _BW, _L), (0, 2, 1))
    sums = _sc_lookup_sum(texts_t, table)
    w_t = jnp.transpose(fc_w) * jnp.float32(1.0 / _L)
    w_pad = jnp.pad(w_t, ((0, 0), (0, 128 - CLASS_NUM)))
    b_pad = jnp.pad(fc_b, (0, 128 - CLASS_NUM)).reshape(1, 128)
    out = _fc_call(sums, w_pad, b_pad)
    return out[:, :CLASS_NUM]


# speedup vs baseline: 1.0657x; 1.0657x over previous
import functools

import jax
import jax.numpy as jnp
from jax import lax
from jax.experimental import pallas as pl
from jax.experimental.pallas import tpu as pltpu
from jax.experimental.pallas import tpu_sc as plsc

VOCAB = 1000000
EMBED_DIM = 64
PAD_LEN = 200
BATCH = 4096
CLASS_NUM = 16

_D = EMBED_DIM
_L = PAD_LEN
_NC = 2
_NS = 16
_NW = _NC * _NS
_BW = BATCH // _NW
_CW = 40
_CPR = _L // _CW
_NCH = _BW * _CPR
_NB = 10


def _fire(table_hbm, tex_v, buf, sem, c):
    off = pl.multiple_of(_CW * c, 8)
    pltpu.make_async_copy(table_hbm.at[tex_v.at[pl.ds(off, _CW)]],
                          buf, sem).start()


def _drain(table_hbm, buf, sem):
    pltpu.make_async_copy(table_hbm.at[pl.ds(0, _CW)], buf, sem).wait()


def _accum(buf, acc_v, row):
    z = jnp.zeros((16,), jnp.float32)

    def body(t, accs):
        a0, a1, a2, a3 = accs
        for u in range(4):
            j = 4 * t + u
            a0 = a0 + buf[j, pl.ds(0, 16)]
            a1 = a1 + buf[j, pl.ds(16, 16)]
            a2 = a2 + buf[j, pl.ds(32, 16)]
            a3 = a3 + buf[j, pl.ds(48, 16)]
        return (a0, a1, a2, a3)

    a0, a1, a2, a3 = lax.fori_loop(0, _CW // 4, body, (z, z, z, z))
    plsc.addupdate(acc_v.at[row, pl.ds(0, 16)], a0)
    plsc.addupdate(acc_v.at[row, pl.ds(16, 16)], a1)
    plsc.addupdate(acc_v.at[row, pl.ds(32, 16)], a2)
    plsc.addupdate(acc_v.at[row, pl.ds(48, 16)], a3)


@functools.partial(
    pl.kernel,
    mesh=plsc.VectorSubcoreMesh(core_axis_name="c", subcore_axis_name="s"),
    out_type=jax.ShapeDtypeStruct((BATCH, _D), jnp.float32),
    compiler_params=pltpu.CompilerParams(use_tc_tiling_on_sc=False),
    scratch_types=(
        [pltpu.VMEM((_BW * _L,), jnp.int32)]
        + [pltpu.VMEM((_BW, _D), jnp.float32)]
        + [pltpu.VMEM((_CW, _D), jnp.float32)] * _NB
        + [pltpu.SemaphoreType.DMA] * _NB
    ),
)
def _sc_lookup_sum(texts_hbm, table_hbm, out_hbm, tex_v, acc_v, *ring):
    bufs, sems = ring[:_NB], ring[_NB:]
    wid = lax.axis_index("s") * _NC + lax.axis_index("c")
    base = wid * _BW
    pltpu.sync_copy(texts_hbm.at[wid], tex_v)

    z = jnp.zeros((16,), jnp.float32)

    def zero_body(t, carry):
        for u in range(16):
            acc_v[4 * t + u // 4, pl.ds(16 * (u % 4), 16)] = z
        return carry
    lax.fori_loop(0, (_BW * _D) // 256, zero_body, 0)

    for b in range(_NB):
        _fire(table_hbm, tex_v, bufs[b], sems[b], b)

    def outer(g, carry):
        for b in range(_NB):
            c = g * _NB + b
            _drain(table_hbm, bufs[b], sems[b])
            _accum(bufs[b], acc_v, c // _CPR)

            @pl.when(c + _NB < _NCH)
            def _():
                _fire(table_hbm, tex_v, bufs[b], sems[b], c + _NB)
        return carry

    lax.fori_loop(0, _NCH // _NB, outer, 0)
    pltpu.sync_copy(acc_v, out_hbm.at[pl.ds(base, _BW)])


def _fc_body(x_ref, w_ref, b_ref, o_ref):
    o_ref[...] = (
        jnp.dot(x_ref[...], w_ref[...], preferred_element_type=jnp.float32)
        + b_ref[...]
    )


_fc_call = pl.pallas_call(
    _fc_body,
    out_shape=jax.ShapeDtypeStruct((BATCH, 128), jnp.float32),
)


def kernel(texts, table, fc_w, fc_b):
    texts_f = texts.astype(jnp.int32).reshape(_NW, _BW * _L)
    sums = _sc_lookup_sum(texts_f, table)
    w_t = jnp.transpose(fc_w) * jnp.float32(1.0 / _L)
    w_pad = jnp.pad(w_t, ((0, 0), (0, 128 - CLASS_NUM)))
    b_pad = jnp.pad(fc_b, (0, 128 - CLASS_NUM)).reshape(1, 128)
    out = _fc_call(sums, w_pad, b_pad)
    return out[:, :CLASS_NUM]
